# baseline (speedup 1.0000x reference)
import functools

import jax
import jax.numpy as jnp
from jax.experimental import pallas as pl
from jax.experimental.pallas import tpu as pltpu

_CH = 32


def _gcn_dma(x_hbm, w1_ref, b1_ref, w2_ref, b2_ref, o_hbm,
             xbuf, obuf, isem, osem):
    B, N, d_in = x_hbm.shape
    d_out = o_hbm.shape[2]
    nc = B // _CH

    def in_copy(c, slot):
        return pltpu.make_async_copy(
            x_hbm.at[pl.ds(c * _CH, _CH)], xbuf.at[slot], isem.at[slot])

    def out_copy(c, slot):
        return pltpu.make_async_copy(
            obuf.at[slot], o_hbm.at[pl.ds(c * _CH, _CH)], osem.at[slot])

    in_copy(0, 0).start()
    in_copy(1, 1).start()
    for c in range(nc):
        slot = c % 2
        in_copy(c, slot).wait()
        x = xbuf[slot]
        m = jnp.sum(x, axis=1) * (1.0 / N)
        h = jnp.dot(m, w1_ref[...], preferred_element_type=jnp.float32)
        h = h + b1_ref[...][None, :]
        y = jnp.dot(h, w2_ref[...], preferred_element_type=jnp.float32)
        y = y + b2_ref[...][None, :]
        if c >= 2:
            out_copy(c - 2, slot).wait()
        obuf[slot] = jnp.broadcast_to(y[:, None, :], (_CH, N, d_out))
        out_copy(c, slot).start()
        if c + 2 < nc:
            in_copy(c + 2, slot).start()
    out_copy(nc - 2, nc % 2).wait()
    out_copy(nc - 1, (nc - 1) % 2).wait()


@functools.partial(jax.jit, static_argnames=())
def kernel(user_batch, W1, b1, W2, b2):
    B, N, d_in = user_batch.shape
    d_hid = W1.shape[1]
    d_out = W2.shape[1]

    return pl.pallas_call(
        _gcn_dma,
        in_specs=[
            pl.BlockSpec(memory_space=pl.ANY),
            pl.BlockSpec(memory_space=pltpu.VMEM),
            pl.BlockSpec(memory_space=pltpu.VMEM),
            pl.BlockSpec(memory_space=pltpu.VMEM),
            pl.BlockSpec(memory_space=pltpu.VMEM),
        ],
        out_specs=pl.BlockSpec(memory_space=pl.ANY),
        out_shape=jax.ShapeDtypeStruct((B, N, d_out), user_batch.dtype),
        scratch_shapes=[
            pltpu.VMEM((2, _CH, N, d_in), jnp.float32),
            pltpu.VMEM((2, _CH, N, d_out), jnp.float32),
            pltpu.SemaphoreType.DMA((2,)),
            pltpu.SemaphoreType.DMA((2,)),
        ],
    )(user_batch, W1, b1, W2, b2)


# speedup vs baseline: 1.1975x; 1.1975x over previous
import functools

import jax
import jax.numpy as jnp
from jax.experimental import pallas as pl
from jax.experimental.pallas import tpu as pltpu

_NC = 4


def _gcn_dma(x_hbm, w1_ref, b1_ref, w2_ref, b2_ref, o_hbm,
             xbuf, obuf, isem, osem):
    B, N, d_in = x_hbm.shape
    d_out = o_hbm.shape[2]
    ch = B // _NC

    def in_copy(c):
        return pltpu.make_async_copy(
            x_hbm.at[pl.ds(c * ch, ch)], xbuf.at[c], isem.at[c])

    def out_copy(c):
        return pltpu.make_async_copy(
            obuf.at[c], o_hbm.at[pl.ds(c * ch, ch)], osem.at[c])

    for c in range(_NC):
        in_copy(c).start()
    for c in range(_NC):
        in_copy(c).wait()
        x = xbuf[c]
        m = jnp.sum(x, axis=1) * (1.0 / N)
        h = jnp.dot(m, w1_ref[...], preferred_element_type=jnp.float32)
        h = h + b1_ref[...][None, :]
        y = jnp.dot(h, w2_ref[...], preferred_element_type=jnp.float32)
        y = y + b2_ref[...][None, :]
        obuf[c] = jnp.broadcast_to(y[:, None, :], (ch, N, d_out))
        out_copy(c).start()
    for c in range(_NC):
        out_copy(c).wait()


@functools.partial(jax.jit, static_argnames=())
def kernel(user_batch, W1, b1, W2, b2):
    B, N, d_in = user_batch.shape
    d_hid = W1.shape[1]
    d_out = W2.shape[1]

    return pl.pallas_call(
        _gcn_dma,
        in_specs=[
            pl.BlockSpec(memory_space=pl.ANY),
            pl.BlockSpec(memory_space=pltpu.VMEM),
            pl.BlockSpec(memory_space=pltpu.VMEM),
            pl.BlockSpec(memory_space=pltpu.VMEM),
            pl.BlockSpec(memory_space=pltpu.VMEM),
        ],
        out_specs=pl.BlockSpec(memory_space=pl.ANY),
        out_shape=jax.ShapeDtypeStruct((B, N, d_out), user_batch.dtype),
        scratch_shapes=[
            pltpu.VMEM((_NC, B // _NC, N, d_in), jnp.float32),
            pltpu.VMEM((_NC, B // _NC, N, d_out), jnp.float32),
            pltpu.SemaphoreType.DMA((_NC,)),
            pltpu.SemaphoreType.DMA((_NC,)),
        ],
    )(user_batch, W1, b1, W2, b2)


# speedup vs baseline: 1.2026x; 1.0043x over previous
import functools

import jax
import jax.numpy as jnp
from jax.experimental import pallas as pl
from jax.experimental.pallas import tpu as pltpu

_NC = 8


def _gcn_dma(x_hbm, w1_ref, b1_ref, w2_ref, b2_ref, o_hbm,
             xbuf, obuf, isem, osem):
    B, N, d_in = x_hbm.shape
    d_out = o_hbm.shape[2]
    ch = B // _NC

    def in_copy(c):
        return pltpu.make_async_copy(
            x_hbm.at[pl.ds(c * ch, ch)], xbuf.at[c], isem.at[c])

    def out_copy(c):
        return pltpu.make_async_copy(
            obuf.at[c], o_hbm.at[pl.ds(c * ch, ch)], osem.at[c])

    for c in range(_NC):
        in_copy(c).start()
    for c in range(_NC):
        in_copy(c).wait()
        x = xbuf[c]
        m = jnp.sum(x, axis=1) * (1.0 / N)
        h = jnp.dot(m, w1_ref[...], preferred_element_type=jnp.float32)
        h = h + b1_ref[...][None, :]
        y = jnp.dot(h, w2_ref[...], preferred_element_type=jnp.float32)
        y = y + b2_ref[...][None, :]
        obuf[c] = jnp.broadcast_to(y[:, None, :], (ch, N, d_out))
        out_copy(c).start()
    for c in range(_NC):
        out_copy(c).wait()


@functools.partial(jax.jit, static_argnames=())
def kernel(user_batch, W1, b1, W2, b2):
    B, N, d_in = user_batch.shape
    d_hid = W1.shape[1]
    d_out = W2.shape[1]

    return pl.pallas_call(
        _gcn_dma,
        in_specs=[
            pl.BlockSpec(memory_space=pl.ANY),
            pl.BlockSpec(memory_space=pltpu.VMEM),
            pl.BlockSpec(memory_space=pltpu.VMEM),
            pl.BlockSpec(memory_space=pltpu.VMEM),
            pl.BlockSpec(memory_space=pltpu.VMEM),
        ],
        out_specs=pl.BlockSpec(memory_space=pl.ANY),
        out_shape=jax.ShapeDtypeStruct((B, N, d_out), user_batch.dtype),
        scratch_shapes=[
            pltpu.VMEM((_NC, B // _NC, N, d_in), jnp.float32),
            pltpu.VMEM((_NC, B // _NC, N, d_out), jnp.float32),
            pltpu.SemaphoreType.DMA((_NC,)),
            pltpu.SemaphoreType.DMA((_NC,)),
        ],
    )(user_batch, W1, b1, W2, b2)


# speedup vs baseline: 1.3513x; 1.1237x over previous
import functools

import jax
import jax.numpy as jnp
from jax.experimental import pallas as pl


def _gcn_block(x_ref, w1_ref, b1_ref, w2_ref, b2_ref, o_ref):
    x = x_ref[...]
    n = x.shape[1]
    m = jnp.sum(x, axis=1) * (1.0 / n)
    h = jnp.dot(m, w1_ref[...], preferred_element_type=jnp.float32)
    h = h + b1_ref[...][None, :]
    y = jnp.dot(h, w2_ref[...], preferred_element_type=jnp.float32)
    y = y + b2_ref[...][None, :]
    o_ref[...] = jnp.broadcast_to(
        y[:, None, :], (x.shape[0], n, y.shape[1])
    )


@functools.partial(jax.jit, static_argnames=())
def kernel(user_batch, W1, b1, W2, b2):
    B, N, d_in = user_batch.shape
    d_hid = W1.shape[1]
    d_out = W2.shape[1]
    Gb = 64

    return pl.pallas_call(
        _gcn_block,
        grid=(B // Gb,),
        in_specs=[
            pl.BlockSpec((Gb, N, d_in), lambda i: (i, 0, 0)),
            pl.BlockSpec((d_in, d_hid), lambda i: (0, 0)),
            pl.BlockSpec((d_hid,), lambda i: (0,)),
            pl.BlockSpec((d_hid, d_out), lambda i: (0, 0)),
            pl.BlockSpec((d_out,), lambda i: (0,)),
        ],
        out_specs=pl.BlockSpec((Gb, N, d_out), lambda i: (i, 0, 0)),
        out_shape=jax.ShapeDtypeStruct((B, N, d_out), user_batch.dtype),
    )(user_batch, W1, b1, W2, b2)
